# Initial kernel scaffold; baseline (speedup 1.0000x reference)
import jax
import jax.numpy as jnp
from jax.experimental import pallas as pl


def kernel(x):
    raise NotImplementedError("write your pallas kernel here")


# speedup vs baseline: 11.2224x; 11.2224x over previous
import jax
import jax.numpy as jnp
from jax.experimental import pallas as pl
from jax.experimental.pallas import tpu as pltpu

_K = 9
_DILATION = 2
_KK = _K * _DILATION
_RANKS = _KK - 1
_RT = 256


def _norm_kernel(x_ref, xn_ref, sq_ref):
    xb = x_ref[0]
    nrm = jnp.sqrt(jnp.sum(xb * xb, axis=1, keepdims=True))
    xn = xb / jnp.maximum(nrm, 1e-12)
    xn_ref[0] = xn
    sq_ref[0, 0] = jnp.sum(xn * xn, axis=1)


def _topk_kernel(a_ref, bt_ref, sq_ref, out_ref):
    a = a_ref[0]
    bt = bt_ref[0]
    inner = jax.lax.dot_general(
        a, bt, (((1,), (0,)), ((), ())),
        preferred_element_type=jnp.float32)
    row_sq = jnp.sum(a * a, axis=1, keepdims=True)
    col_sq = sq_ref[0, 0, :][None, :]
    dist = (row_sq + (-2.0) * inner) + col_sq
    val = -dist
    iota = jax.lax.broadcasted_iota(jnp.int32, val.shape, 1)
    big = jnp.int32(2 ** 30)
    idxs = []
    for _ in range(_RANKS):
        m = jnp.max(val, axis=1, keepdims=True)
        hit = val == m
        idx = jnp.min(jnp.where(hit, iota, big), axis=1)
        idxs.append(idx)
        val = jnp.where(iota == idx[:, None], -jnp.inf, val)
    out_ref[0] = jnp.stack(idxs[0::2], axis=1)


def kernel(x):
    B, C, N, _ = x.shape
    xt = jnp.squeeze(jnp.transpose(x, (0, 2, 1, 3)), -1)

    xn, sq = pl.pallas_call(
        _norm_kernel,
        grid=(B,),
        in_specs=[pl.BlockSpec((1, N, C), lambda b: (b, 0, 0))],
        out_specs=[
            pl.BlockSpec((1, N, C), lambda b: (b, 0, 0)),
            pl.BlockSpec((1, 1, N), lambda b: (b, 0, 0)),
        ],
        out_shape=[
            jax.ShapeDtypeStruct((B, N, C), jnp.float32),
            jax.ShapeDtypeStruct((B, 1, N), jnp.float32),
        ],
    )(xt)

    xnt = jnp.transpose(xn, (0, 2, 1))

    nn = pl.pallas_call(
        _topk_kernel,
        grid=(B, N // _RT),
        in_specs=[
            pl.BlockSpec((1, _RT, C), lambda b, i: (b, i, 0)),
            pl.BlockSpec((1, C, N), lambda b, i: (b, 0, 0)),
            pl.BlockSpec((1, 1, N), lambda b, i: (b, 0, 0)),
        ],
        out_specs=pl.BlockSpec((1, _RT, _K), lambda b, i: (b, i, 0)),
        out_shape=jax.ShapeDtypeStruct((B, N, _K), jnp.int32),
    )(xn, xnt, sq)

    center = jnp.broadcast_to(
        jnp.arange(N, dtype=nn.dtype)[None, :, None], (B, N, _K))
    return jnp.stack((nn, center), axis=0)


# speedup vs baseline: 13.8307x; 1.2324x over previous
import jax
import jax.numpy as jnp
from jax.experimental import pallas as pl
from jax.experimental.pallas import tpu as pltpu

_K = 9
_DILATION = 2
_KK = _K * _DILATION
_RANKS = _KK - 1
_RT = 256


def _norm_kernel(x_ref, xn_ref, sq_ref):
    xb = x_ref[0]
    nrm = jnp.sqrt(jnp.sum(xb * xb, axis=0, keepdims=True))
    xn = xb / jnp.maximum(nrm, 1e-12)
    xn_ref[0] = xn
    sq_ref[0, 0] = jnp.sum(xn * xn, axis=0)


def _topk_kernel(a_ref, b_ref, sq_ref, out_ref):
    i = pl.program_id(1)
    a = a_ref[0]
    b = b_ref[0]
    inner = jax.lax.dot_general(
        a, b, (((0,), (0,)), ((), ())),
        preferred_element_type=jnp.float32)
    row_sq = sq_ref[0, 0, pl.ds(i * _RT, _RT)][:, None]
    col_sq = sq_ref[0, 0, :][None, :]
    dist = (row_sq + (-2.0) * inner) + col_sq
    val = -dist
    n = val.shape[1]
    iiota = jax.lax.broadcasted_iota(jnp.int32, val.shape, 1)
    fiota = (jnp.int32(n) - iiota).astype(jnp.float32)
    neg = -jnp.inf
    revs = []
    for _ in range(_RANKS):
        m = jnp.max(val, axis=1, keepdims=True)
        cand = jnp.where(val == m, fiota, neg)
        fr = jnp.max(cand, axis=1, keepdims=True)
        val = jnp.where(cand == fr, neg, val)
        revs.append(fr[:, 0])
    ranks = jnp.stack(revs[0::2], axis=1)
    out_ref[0] = (jnp.float32(n) - ranks).astype(jnp.int32)


def kernel(x):
    B, C, N, _ = x.shape
    xc = x[..., 0]

    xn, sq = pl.pallas_call(
        _norm_kernel,
        grid=(B,),
        in_specs=[pl.BlockSpec((1, C, N), lambda b: (b, 0, 0))],
        out_specs=[
            pl.BlockSpec((1, C, N), lambda b: (b, 0, 0)),
            pl.BlockSpec((1, 1, N), lambda b: (b, 0, 0)),
        ],
        out_shape=[
            jax.ShapeDtypeStruct((B, C, N), jnp.float32),
            jax.ShapeDtypeStruct((B, 1, N), jnp.float32),
        ],
    )(xc)

    nn = pl.pallas_call(
        _topk_kernel,
        grid=(B, N // _RT),
        in_specs=[
            pl.BlockSpec((1, C, _RT), lambda b, i: (b, 0, i)),
            pl.BlockSpec((1, C, N), lambda b, i: (b, 0, 0)),
            pl.BlockSpec((1, 1, N), lambda b, i: (b, 0, 0)),
        ],
        out_specs=pl.BlockSpec((1, _RT, _K), lambda b, i: (b, i, 0)),
        out_shape=jax.ShapeDtypeStruct((B, N, _K), jnp.int32),
    )(xn, xn, sq)

    center = jnp.broadcast_to(
        jnp.arange(N, dtype=nn.dtype)[None, :, None], (B, N, _K))
    return jnp.stack((nn, center), axis=0)


# speedup vs baseline: 13.9080x; 1.0056x over previous
import jax
import jax.numpy as jnp
from jax.experimental import pallas as pl
from jax.experimental.pallas import tpu as pltpu

_K = 9
_DILATION = 2
_KK = _K * _DILATION
_RANKS = _KK - 1
_RT = 512


def _norm_kernel(x_ref, xn_ref, sq_ref):
    xb = x_ref[0]
    nrm = jnp.sqrt(jnp.sum(xb * xb, axis=0, keepdims=True))
    xn = xb / jnp.maximum(nrm, 1e-12)
    xn_ref[0] = xn
    sq_ref[0, 0] = jnp.sum(xn * xn, axis=0)


def _topk_kernel(a_ref, b_ref, sq_ref, out_ref):
    i = pl.program_id(1)
    a = a_ref[0]
    b = b_ref[0]
    inner = jax.lax.dot_general(
        a, b, (((0,), (0,)), ((), ())),
        preferred_element_type=jnp.float32)
    row_sq = sq_ref[0, 0, pl.ds(i * _RT, _RT)][:, None]
    col_sq = sq_ref[0, 0, :][None, :]
    dist = (row_sq + (-2.0) * inner) + col_sq
    val = -dist
    n = val.shape[1]
    iiota = jax.lax.broadcasted_iota(jnp.int32, val.shape, 1)
    fiota = (jnp.int32(n) - iiota).astype(jnp.float32)
    neg = -jnp.inf
    revs = []
    for step in range(_RANKS):
        m = jnp.max(val, axis=1, keepdims=True)
        cand = jnp.where(val == m, fiota, neg)
        fr = jnp.max(cand, axis=1, keepdims=True)
        if step != _RANKS - 1:
            val = jnp.where(cand == fr, neg, val)
        revs.append(fr[:, 0])
    ranks = jnp.stack(revs[0::2], axis=1)
    out_ref[0] = (jnp.float32(n) - ranks).astype(jnp.int32)


def kernel(x):
    B, C, N, _ = x.shape
    xc = x[..., 0]

    xn, sq = pl.pallas_call(
        _norm_kernel,
        grid=(B,),
        in_specs=[pl.BlockSpec((1, C, N), lambda b: (b, 0, 0))],
        out_specs=[
            pl.BlockSpec((1, C, N), lambda b: (b, 0, 0)),
            pl.BlockSpec((1, 1, N), lambda b: (b, 0, 0)),
        ],
        out_shape=[
            jax.ShapeDtypeStruct((B, C, N), jnp.float32),
            jax.ShapeDtypeStruct((B, 1, N), jnp.float32),
        ],
    )(xc)

    nn = pl.pallas_call(
        _topk_kernel,
        grid=(B, N // _RT),
        in_specs=[
            pl.BlockSpec((1, C, _RT), lambda b, i: (b, 0, i)),
            pl.BlockSpec((1, C, N), lambda b, i: (b, 0, 0)),
            pl.BlockSpec((1, 1, N), lambda b, i: (b, 0, 0)),
        ],
        out_specs=pl.BlockSpec((1, _RT, _K), lambda b, i: (b, i, 0)),
        out_shape=jax.ShapeDtypeStruct((B, N, _K), jnp.int32),
    )(xn, xn, sq)

    center = jnp.broadcast_to(
        jnp.arange(N, dtype=nn.dtype)[None, :, None], (B, N, _K))
    return jnp.stack((nn, center), axis=0)


# speedup vs baseline: 13.9085x; 1.0000x over previous
import jax
import jax.numpy as jnp
from jax.experimental import pallas as pl

_K = 9
_DILATION = 2
_KK = _K * _DILATION
_RANKS = _KK - 1
_RT = 512


def _norm_kernel(x_ref, xn_ref, sq_ref):
    xb = x_ref[0]
    nrm = jnp.sqrt(jnp.sum(xb * xb, axis=0, keepdims=True))
    xn = xb / jnp.maximum(nrm, 1e-12)
    xn_ref[0] = xn
    sq_ref[0, 0] = jnp.sum(xn * xn, axis=0)


def _topk_kernel(a_ref, b_ref, sq_ref, out_ref):
    i = pl.program_id(1)
    a = a_ref[0]
    b = b_ref[0]
    inner = jax.lax.dot_general(
        a, b, (((0,), (0,)), ((), ())),
        preferred_element_type=jnp.float32)
    row_sq = sq_ref[0, 0, pl.ds(i * _RT, _RT)][:, None]
    col_sq = sq_ref[0, 0, :][None, :]
    dist = (row_sq + (-2.0) * inner) + col_sq
    val = -dist
    n = val.shape[1]
    iiota = jax.lax.broadcasted_iota(jnp.int32, val.shape, 1)
    fiota = (jnp.int32(n) - iiota).astype(jnp.float32)
    neg = -jnp.inf
    revs = []
    for step in range(_RANKS):
        m = jnp.max(val, axis=1, keepdims=True)
        cand = jnp.where(val == m, fiota, neg)
        fr = jnp.max(cand, axis=1, keepdims=True)
        if step != _RANKS - 1:
            val = jnp.where(cand == fr, neg, val)
        revs.append(fr[:, 0])
    ranks = jnp.stack(revs[0::2], axis=1)
    out_ref[0] = (jnp.float32(n) - ranks).astype(jnp.int32)


def kernel(x):
    B, C, N, _ = x.shape
    xc = x[..., 0]

    xn, sq = pl.pallas_call(
        _norm_kernel,
        grid=(B,),
        in_specs=[pl.BlockSpec((1, C, N), lambda b: (b, 0, 0))],
        out_specs=[
            pl.BlockSpec((1, C, N), lambda b: (b, 0, 0)),
            pl.BlockSpec((1, 1, N), lambda b: (b, 0, 0)),
        ],
        out_shape=[
            jax.ShapeDtypeStruct((B, C, N), jnp.float32),
            jax.ShapeDtypeStruct((B, 1, N), jnp.float32),
        ],
    )(xc)

    nn = pl.pallas_call(
        _topk_kernel,
        grid=(B, N // _RT),
        in_specs=[
            pl.BlockSpec((1, C, _RT), lambda b, i: (b, 0, i)),
            pl.BlockSpec((1, C, N), lambda b, i: (b, 0, 0)),
            pl.BlockSpec((1, 1, N), lambda b, i: (b, 0, 0)),
        ],
        out_specs=pl.BlockSpec((1, _RT, _K), lambda b, i: (b, i, 0)),
        out_shape=jax.ShapeDtypeStruct((B, N, _K), jnp.int32),
    )(xn, xn, sq)

    center = jnp.broadcast_to(
        jnp.arange(N, dtype=nn.dtype)[None, :, None], (B, N, _K))
    return jnp.stack((nn, center), axis=0)
